# Initial kernel scaffold; baseline (speedup 1.0000x reference)
import jax
import jax.numpy as jnp
from jax.experimental import pallas as pl


def kernel(feats_A, pca_C2, rni, params, edge_B1, edge_B2, dst_G1, edge_I2):
    raise NotImplementedError("write your pallas kernel here")


# speedup vs baseline: 1.0000x; 1.0000x over previous
import jax
import jax.numpy as jnp
from jax.experimental import pallas as pl
from jax.experimental.pallas import tpu as pltpu

N_A = 10000
N_C2 = 2000
HID = 128
HEADS = 4


def _mlp_bn(x, W1, b1, g1, be1, W2, b2):
    h = x @ W1 + b1
    mu = jnp.mean(h, axis=0)
    var = jnp.var(h, axis=0)
    h = (h - mu) / jnp.sqrt(var + 1e-5) * g1 + be1
    h = jax.nn.relu(h)
    return h @ W2 + b2


def _gat(x, src, dst, N, W, al, ar, b):
    H, D = al.shape
    feat = (x @ W).reshape(-1, H, D)
    el = jnp.sum(feat * al, axis=-1)
    er = jnp.sum(feat * ar, axis=-1)
    e = jax.nn.leaky_relu(el[src] + er[dst], negative_slope=0.2)
    m = jax.ops.segment_max(e, dst, num_segments=N)
    m = jnp.where(jnp.isfinite(m), m, 0.0)
    ex = jnp.exp(e - m[dst])
    den = jax.ops.segment_sum(ex, dst, num_segments=N)
    alpha = ex / jnp.maximum(den[dst], 1e-9)
    out = jax.ops.segment_sum(alpha[:, :, None] * feat[src], dst, num_segments=N)
    return jax.nn.relu(out.reshape(N, H * D) + b)


def _gin(x, src, dst, N, eps, W1, b1, g1, be1, W2, b2):
    agg = jax.ops.segment_sum(x[src], dst, num_segments=N)
    return jax.nn.relu(_mlp_bn((1.0 + eps) * x + agg, W1, b1, g1, be1, W2, b2))


def _out_mlp_kernel(hh_ref, w1_ref, b1_ref, w2_ref, b2_ref, o_ref):
    h = jnp.maximum(
        jnp.matmul(hh_ref[:], w1_ref[:], precision='highest') + b1_ref[:], 0.0)
    o_ref[:] = jnp.matmul(h, w2_ref[:], precision='highest') + b2_ref[:]


def kernel(feats_A, pca_C2, rni, params, edge_B1, edge_B2, dst_G1, edge_I2):
    p = params
    feats = jnp.concatenate([feats_A, rni], axis=-1)
    hs = []
    for t, ei in enumerate([edge_B1, edge_B2]):
        src, dst = ei[0], ei[1]
        h = _gat(feats, src, dst, N_A, p['gat%d_W' % t], p['gat%d_al' % t],
                 p['gat%d_ar' % t], p['gat%d_b' % t])
        h = jnp.concatenate([h, feats], axis=-1)
        h = _gin(h, src, dst, N_A, 0.0, p['gin%d_W1' % t], p['gin%d_b1' % t],
                 p['gin%d_g1' % t], p['gin%d_be1' % t], p['gin%d_W2' % t],
                 p['gin%d_b2' % t])
        hs.append(h)
    hA = jnp.concatenate(hs, axis=1)
    hC = jax.ops.segment_max(hA, dst_G1, num_segments=N_C2)
    hC = jnp.where(jnp.isfinite(hC), hC, 0.0)
    h = jnp.concatenate([hC, pca_C2[..., :4]], axis=-1)
    src2, dst2 = edge_I2[0], edge_I2[1]
    for i in range(2):
        h = _gin(h, src2, dst2, N_C2, 0.0, p['h2c%d_W1' % i], p['h2c%d_b1' % i],
                 p['h2c%d_g1' % i], p['h2c%d_be1' % i], p['h2c%d_W2' % i],
                 p['h2c%d_b2' % i])
    h1 = jnp.mean(hA, axis=0, keepdims=True)
    h2 = jnp.mean(h, axis=0, keepdims=True)
    hh = jnp.concatenate([h1, h2], axis=-1)
    o = pl.pallas_call(
        _out_mlp_kernel,
        out_shape=jax.ShapeDtypeStruct((1, 1), jnp.float32),
    )(hh, p['out_W1'], p['out_b1'][None, :], p['out_W2'], p['out_b2'][None, :])
    return o


# speedup vs baseline: 26.9546x; 26.9545x over previous
import jax
import jax.numpy as jnp
import numpy as np
from jax import lax
from jax.experimental import pallas as pl
from jax.experimental.pallas import tpu as pltpu
from jax.experimental.pallas import tpu_sc as plsc

N_A = 10000
N_C2 = 2000
HID = 128
HEADS = 4
E_B = 320000
E_I2 = 32000

NC = 2
NS = 16

_CP = pltpu.CompilerParams(use_tc_tiling_on_sc=False, needs_layout_passes=False)


def _mesh():
    return plsc.VectorSubcoreMesh(core_axis_name="c", subcore_axis_name="s",
                                  num_cores=NC, num_subcores=NS)


def _zero16(ref, n):
    cgrp = ref.shape[1] // 16

    def it(i, _):
        ref[i // cgrp, pl.ds((i % cgrp) * 16, 16)] = jnp.zeros((16,), jnp.float32)
        return 0
    lax.fori_loop(0, n * cgrp, it, 0)


def _make_sc_rowsum(N, C, E, B, name):
    assert C % 16 == 0 and (C * 4) % 64 == 0 and N % 16 == 0
    assert E % (NC * NS * B) == 0 and B <= 128 and B % 8 == 0
    nchunks = N // 16
    iters = (nchunks + NS - 1) // NS
    e_per_tile = E // (NC * NS)
    nb = e_per_tile // B

    def body(table, src, dst, out, acc, zv, sv, dv, rows, sem):
        c = lax.axis_index("c")
        s = lax.axis_index("s")
        _zero16(zv, 16)

        def zchunk(i, _):
            k = i * NS + s

            @pl.when(k < nchunks)
            def _():
                pltpu.sync_copy(zv, acc.at[pl.ds(k * 16, 16)])
            return 0
        lax.fori_loop(0, iters, zchunk, 0)
        plsc.subcore_barrier()

        def batch(j, _):
            e0 = c * (E // NC) + s * e_per_tile + j * B
            pltpu.sync_copy(src.at[pl.ds(e0, B)], sv)
            pltpu.sync_copy(dst.at[pl.ds(e0, B)], dv)
            pltpu.async_copy(table.at[sv], rows, sem).wait()
            pltpu.sync_copy(rows, acc.at[dv], add=True)
            return 0
        lax.fori_loop(0, nb, batch, 0)
        plsc.subcore_barrier()

        def wchunk(i, _):
            k = i * NS + s

            @pl.when(k < nchunks)
            def _():
                pltpu.sync_copy(acc.at[pl.ds(k * 16, 16)],
                                out.at[c, pl.ds(k * 16, 16)])
            return 0
        lax.fori_loop(0, iters, wchunk, 0)

    return pl.kernel(
        body,
        out_type=jax.ShapeDtypeStruct((NC, N, C), jnp.float32),
        mesh=_mesh(),
        compiler_params=_CP,
        scratch_types=[
            pltpu.VMEM_SHARED((N, C), jnp.float32),
            pltpu.VMEM((16, C), jnp.float32),
            pltpu.VMEM((B,), jnp.int32),
            pltpu.VMEM((B,), jnp.int32),
            pltpu.VMEM((B, C), jnp.float32),
            pltpu.SemaphoreType.DMA,
        ],
        name=name,
    )


def _make_sc_gat(E, B, name):
    assert E % (NS * B) == 0 and B % 16 == 0
    e_per_tile = E // NS
    nb = e_per_tile // B
    nchunks = N_A // 16
    iters = (nchunks + NS - 1) // NS

    def body(X, eler, srcc, dstc, U, den,
             accU, accD, zv, sv, dv, elsrc, eldst, exb, rows, smX, smS, smD):
        c = lax.axis_index("c")
        s = lax.axis_index("s")
        iot = lax.iota(jnp.int32, 16)
        _zero16(zv, 16)

        def zex(i, _):
            exb[i, :] = jnp.zeros((16,), jnp.float32)
            return 0
        lax.fori_loop(0, B, zex, 0)

        def zchunk(i, _):
            k = i * NS + s

            @pl.when(k < nchunks)
            def _():
                pltpu.sync_copy(zv, accU.at[pl.ds(k * 16, 16)])
                pltpu.sync_copy(zv.at[:, pl.ds(0, 16)],
                                accD.at[pl.ds(k * 16, 16)])
            return 0
        lax.fori_loop(0, iters, zchunk, 0)
        plsc.subcore_barrier()

        def batch(j, _):
            e0 = c * E + s * e_per_tile + j * B
            pltpu.sync_copy(srcc.at[pl.ds(e0, B)], sv)
            pltpu.sync_copy(dstc.at[pl.ds(e0, B)], dv)
            cpX = pltpu.async_copy(X.at[c].at[sv], rows, smX)
            cpS = pltpu.async_copy(eler.at[c].at[sv], elsrc, smS)
            cpD = pltpu.async_copy(eler.at[c].at[dv], eldst, smD)
            cpX.wait()
            cpS.wait()
            cpD.wait()
            for i5 in range(B // 16):
                ridx = i5 * 16 + iot
                for h in range(4):
                    el_s = plsc.load_gather(
                        elsrc, [ridx, jnp.full((16,), h, jnp.int32)])
                    er_d = plsc.load_gather(
                        eldst, [ridx, jnp.full((16,), h + 4, jnp.int32)])
                    z = el_s + er_d
                    z = jnp.where(z >= 0.0, z, 0.2 * z)
                    plsc.store_scatter(
                        exb, [ridx, jnp.full((16,), h, jnp.int32)], jnp.exp(z))
            pltpu.sync_copy(exb, accD.at[dv], add=True)

            def scale(i, _):
                for h in range(4):
                    sc = plsc.load_gather(exb, [jnp.full((16,), i, jnp.int32),
                                                jnp.full((16,), h, jnp.int32)])
                    for gg in (2 * h, 2 * h + 1):
                        v = rows[i, pl.ds(gg * 16, 16)]
                        rows[i, pl.ds(gg * 16, 16)] = v * sc
                return 0
            lax.fori_loop(0, B, scale, 0)
            pltpu.sync_copy(rows, accU.at[dv], add=True)
            return 0
        lax.fori_loop(0, nb, batch, 0)
        plsc.subcore_barrier()

        def wchunk(i, _):
            k = i * NS + s

            @pl.when(k < nchunks)
            def _():
                pltpu.sync_copy(accU.at[pl.ds(k * 16, 16)],
                                U.at[c, pl.ds(k * 16, 16)])
                pltpu.sync_copy(accD.at[pl.ds(k * 16, 16)],
                                den.at[c, pl.ds(k * 16, 16)])
            return 0
        lax.fori_loop(0, iters, wchunk, 0)

    return pl.kernel(
        body,
        out_type=(jax.ShapeDtypeStruct((NC, N_A, 128), jnp.float32),
                  jax.ShapeDtypeStruct((NC, N_A, 16), jnp.float32)),
        mesh=_mesh(),
        compiler_params=_CP,
        scratch_types=[
            pltpu.VMEM_SHARED((N_A, 128), jnp.float32),
            pltpu.VMEM_SHARED((N_A, 16), jnp.float32),
            pltpu.VMEM((16, 128), jnp.float32),
            pltpu.VMEM((B,), jnp.int32),
            pltpu.VMEM((B,), jnp.int32),
            pltpu.VMEM((B, 16), jnp.float32),
            pltpu.VMEM((B, 16), jnp.float32),
            pltpu.VMEM((B, 16), jnp.float32),
            pltpu.VMEM((B, 128), jnp.float32),
            pltpu.SemaphoreType.DMA,
            pltpu.SemaphoreType.DMA,
            pltpu.SemaphoreType.DMA,
        ],
        name=name,
    )


def _make_sc_segmax(name):
    def body(h, dstg, out, colsv, dvv, acc):
        c = lax.axis_index("c")
        s = lax.axis_index("s")
        iot = lax.iota(jnp.int32, 16)
        msk = iot < 8
        pltpu.sync_copy(h.at[c, :, pl.ds(8 * s, 8)], colsv)
        pltpu.sync_copy(dstg.at[pl.ds(0, N_A)], dvv)

        def z(i, _):
            acc[i, :] = jnp.zeros((16,), jnp.float32)
            return 0
        lax.fori_loop(0, N_C2, z, 0)

        def it(i, _):
            ivec = jnp.full((16,), i, jnp.int32)
            dvec = plsc.load_gather(dvv, [ivec])
            dval = plsc.load_gather(colsv, [ivec, iot], mask=msk)
            av = plsc.load_gather(acc, [dvec, iot], mask=msk)
            plsc.store_scatter(acc, [dvec, iot], jnp.maximum(av, dval), mask=msk)
            return 0
        lax.fori_loop(0, N_A, it, 0)
        pltpu.sync_copy(acc.at[:, pl.ds(0, 8)], out.at[c, :, pl.ds(8 * s, 8)])

    return pl.kernel(
        body,
        out_type=jax.ShapeDtypeStruct((NC, N_C2, 128), jnp.float32),
        mesh=_mesh(),
        compiler_params=_CP,
        scratch_types=[
            pltpu.VMEM((N_A, 8), jnp.float32),
            pltpu.VMEM((N_A,), jnp.int32),
            pltpu.VMEM((N_C2, 16), jnp.float32),
        ],
        name=name,
    )


_sc_rowsum_A = _make_sc_rowsum(N_A, 128, E_B, 80, "sc_rowsum_a")
_sc_rowsum_C272 = _make_sc_rowsum(N_C2, 272, E_I2, 40, "sc_rowsum_c272")
_sc_rowsum_C128 = _make_sc_rowsum(N_C2, 128, E_I2, 40, "sc_rowsum_c128")
_sc_gat = _make_sc_gat(E_B, 80, "sc_gat")
_sc_segmax = _make_sc_segmax("sc_segmax")


def _out_mlp_kernel(hh_ref, w1_ref, b1_ref, w2_ref, b2_ref, o_ref):
    h = jnp.maximum(
        jnp.matmul(hh_ref[:], w1_ref[:], precision='highest') + b1_ref[:], 0.0)
    o_ref[:] = jnp.matmul(h, w2_ref[:], precision='highest') + b2_ref[:]


def kernel(feats_A, pca_C2, rni, params, edge_B1, edge_B2, dst_G1, edge_I2):
    p = params
    f32 = jnp.float32

    feats = jnp.concatenate([feats_A, rni], axis=-1)
    srccat = jnp.concatenate([edge_B1[0], edge_B2[0]])
    dstcat = jnp.concatenate([edge_B1[1], edge_B2[1]])

    Xs, elers = [], []
    for t in range(2):
        al, ar = p['gat%d_al' % t], p['gat%d_ar' % t]
        Xt = jnp.matmul(feats, p['gat%d_W' % t], precision='highest')
        ft = Xt.reshape(N_A, 4, 32)
        el = jnp.sum(ft * al, axis=-1)
        er = jnp.sum(ft * ar, axis=-1)
        Xs.append(Xt)
        elers.append(jnp.pad(jnp.concatenate([el, er], axis=1), ((0, 0), (0, 8))))
    X = jnp.stack(Xs)
    eler = jnp.stack(elers)

    U, den = _sc_gat(X, eler, srccat, dstcat)

    hs, hsums = [], []
    for t, ei in enumerate([edge_B1, edge_B2]):
        src, dst = ei[0], ei[1]
        dent = jnp.maximum(den[t][:, :4], 1e-9)
        alpha_rec = jnp.repeat(1.0 / dent, 32, axis=1)
        hgat_t = jax.nn.relu(U[t] * alpha_rec + p['gat%d_b' % t])
        gp = _sc_rowsum_A(hgat_t, src, dst)
        fp = _sc_rowsum_A(feats, src, dst)
        x = jnp.concatenate([hgat_t, feats], axis=-1)
        agg = jnp.concatenate([gp[0] + gp[1], fp[0] + fp[1]], axis=-1)
        z = x + agg
        h = jnp.matmul(z, p['gin%d_W1' % t], precision='highest') + p['gin%d_b1' % t]
        mu = jnp.mean(h, axis=0)
        d = h - mu
        var = jnp.mean(d * d, axis=0)
        h = d / jnp.sqrt(var + 1e-5) * p['gin%d_g1' % t] + p['gin%d_be1' % t]
        h = jax.nn.relu(h)
        h = jnp.matmul(h, p['gin%d_W2' % t], precision='highest') + p['gin%d_b2' % t]
        h = jax.nn.relu(h)
        hs.append(h)
        hsums.append(jnp.sum(h, axis=0, keepdims=True))

    hstack = jnp.stack(hs)
    hC = _sc_segmax(hstack, dst_G1)

    h272 = jnp.concatenate(
        [hC[0], hC[1], pca_C2[..., :4], jnp.zeros((N_C2, 12), f32)], axis=1)
    ap = _sc_rowsum_C272(h272, edge_I2[0], edge_I2[1])
    x = (h272 + ap[0] + ap[1])[:, :260]
    h = jnp.matmul(x, p['h2c0_W1'], precision='highest') + p['h2c0_b1']
    mu = jnp.mean(h, axis=0); d = h - mu; var = jnp.mean(d * d, axis=0)
    h = jax.nn.relu(d / jnp.sqrt(var + 1e-5) * p['h2c0_g1'] + p['h2c0_be1'])
    g = jax.nn.relu(jnp.matmul(h, p['h2c0_W2'], precision='highest') + p['h2c0_b2'])

    ap2 = _sc_rowsum_C128(g, edge_I2[0], edge_I2[1])
    x = g + ap2[0] + ap2[1]
    h = jnp.matmul(x, p['h2c1_W1'], precision='highest') + p['h2c1_b1']
    mu = jnp.mean(h, axis=0); d = h - mu; var = jnp.mean(d * d, axis=0)
    h = jax.nn.relu(d / jnp.sqrt(var + 1e-5) * p['h2c1_g1'] + p['h2c1_be1'])
    h2 = jax.nn.relu(jnp.matmul(h, p['h2c1_W2'], precision='highest') + p['h2c1_b2'])

    h1m = jnp.concatenate(hsums, axis=0).reshape(1, 256) * (1.0 / N_A)
    h2m = jnp.sum(h2, axis=0, keepdims=True) * (1.0 / N_C2)
    hh = jnp.concatenate([h1m, h2m], axis=-1)
    o = pl.pallas_call(
        _out_mlp_kernel,
        out_shape=jax.ShapeDtypeStruct((1, 1), f32),
    )(hh, p['out_W1'], p['out_b1'][None, :], p['out_W2'], p['out_b2'][None, :])
    return o
